# baseline (speedup 1.0000x reference)
import functools

import jax
import jax.numpy as jnp
from jax import lax
from jax.experimental import pallas as pl
from jax.experimental.pallas import tpu as pltpu
from jax.experimental.pallas import tpu_sc as plsc

B, N, D = 1024, 77, 768
R = B * N
NC, NS = 2, 16
NW = NC * NS
CPW = R // NW
NV = CPW // 16


def _sc_mask_body(tok_hbm, pt_hbm, mask_hbm, tok_v, pt_v, m_v):
    wid = lax.axis_index("s") * NC + lax.axis_index("c")
    base = wid * CPW
    pltpu.sync_copy(tok_hbm.at[pl.ds(base, CPW)], tok_v)
    pltpu.sync_copy(pt_hbm, pt_v)
    ptv = pt_v[...]

    def body(i, carry):
        tv = tok_v[pl.ds(i * 16, 16)]
        m_v[pl.ds(i * 16, 16)] = jnp.where(tv == ptv, 1.0, 0.0)
        return carry

    lax.fori_loop(0, NV, body, 0)
    pltpu.sync_copy(m_v, mask_hbm.at[pl.ds(base, CPW)])


def _select_body(m_ref, emb_ref, ph_ref, out_ref):
    m = m_ref[0]
    out_ref[0] = jnp.where(m != 0.0, ph_ref[0], emb_ref[0])


def kernel(tokenized_text, embedded_text, placeholder_embedding, placeholder_token):
    tok_flat = tokenized_text.T.reshape(R)
    pt16 = jnp.full((16,), placeholder_token, tokenized_text.dtype)
    emb_t = embedded_text.transpose(1, 0, 2)
    ph3 = placeholder_embedding[None]

    sc_mask = functools.partial(
        pl.kernel,
        out_type=jax.ShapeDtypeStruct((R,), jnp.float32),
        mesh=plsc.VectorSubcoreMesh(core_axis_name="c", subcore_axis_name="s"),
        scratch_types=[
            pltpu.VMEM((CPW,), jnp.int32),
            pltpu.VMEM((16,), jnp.int32),
            pltpu.VMEM((CPW,), jnp.float32),
        ],
    )(_sc_mask_body)
    mask3 = sc_mask(tok_flat, pt16).reshape(N, B, 1)

    out_t = pl.pallas_call(
        _select_body,
        grid=(N,),
        in_specs=[
            pl.BlockSpec((1, B, 1), lambda j: (j, 0, 0)),
            pl.BlockSpec((1, B, D), lambda j: (j, 0, 0)),
            pl.BlockSpec((1, 1, D), lambda j: (0, 0, 0)),
        ],
        out_specs=pl.BlockSpec((1, B, D), lambda j: (j, 0, 0)),
        out_shape=jax.ShapeDtypeStruct((N, B, D), jnp.float32),
        compiler_params=pltpu.CompilerParams(
            dimension_semantics=("arbitrary",),
        ),
    )(mask3, emb_t, ph3)
    return out_t.transpose(1, 0, 2)


# speedup vs baseline: 1.2126x; 1.2126x over previous
import functools

import jax
import jax.numpy as jnp
from jax import lax
from jax.experimental import pallas as pl
from jax.experimental.pallas import tpu as pltpu
from jax.experimental.pallas import tpu_sc as plsc

B, N, D = 1024, 77, 768
R = B * N
NC, NS = 2, 16
NW = NC * NS
CPW = R // NW
NV = CPW // 16


def _mask_body(pt_ref, tok_ref, mask_ref):
    mask_ref[...] = jnp.where(tok_ref[...] == pt_ref[0], 1.0, 0.0)


def _select_body(m_ref, emb_ref, ph_ref, out_ref):
    m = m_ref[0]
    out_ref[0] = jnp.where(m != 0.0, ph_ref[0], emb_ref[0])


def kernel(tokenized_text, embedded_text, placeholder_embedding, placeholder_token):
    tok_t = tokenized_text.T
    pt = placeholder_token.reshape((1,)).astype(tokenized_text.dtype)
    emb_t = embedded_text.transpose(1, 0, 2)
    ph3 = placeholder_embedding[None]

    mask2 = pl.pallas_call(
        _mask_body,
        grid_spec=pltpu.PrefetchScalarGridSpec(
            num_scalar_prefetch=1,
            grid=(1,),
            in_specs=[pl.BlockSpec((N, B), lambda i, pt: (0, 0))],
            out_specs=pl.BlockSpec((N, B), lambda i, pt: (0, 0)),
        ),
        out_shape=jax.ShapeDtypeStruct((N, B), jnp.float32),
    )(pt, tok_t)
    mask3 = mask2.reshape(N, B, 1)

    out_t = pl.pallas_call(
        _select_body,
        grid=(N,),
        in_specs=[
            pl.BlockSpec((1, B, 1), lambda j: (j, 0, 0)),
            pl.BlockSpec((1, B, D), lambda j: (j, 0, 0)),
            pl.BlockSpec((1, 1, D), lambda j: (0, 0, 0)),
        ],
        out_specs=pl.BlockSpec((1, B, D), lambda j: (j, 0, 0)),
        out_shape=jax.ShapeDtypeStruct((N, B, D), jnp.float32),
        compiler_params=pltpu.CompilerParams(
            dimension_semantics=("arbitrary",),
        ),
    )(mask3, emb_t, ph3)
    return out_t.transpose(1, 0, 2)


# speedup vs baseline: 1.4243x; 1.1746x over previous
import functools

import jax
import jax.numpy as jnp
from jax import lax
from jax.experimental import pallas as pl
from jax.experimental.pallas import tpu as pltpu

B, N, D = 1024, 77, 768


def _select_body(pt_ref, tok_ref, emb_ref, ph_ref, out_ref):
    j = pl.program_id(0)
    tok = tok_ref[...]
    lane = lax.broadcasted_iota(jnp.int32, (B, N), 1)
    hit = jnp.where((tok == pt_ref[0]) & (lane == j), 1, 0)
    col = jnp.max(hit, axis=1, keepdims=True)
    out_ref[0] = jnp.where(col == 1, ph_ref[0], emb_ref[0])


def kernel(tokenized_text, embedded_text, placeholder_embedding, placeholder_token):
    pt = placeholder_token.reshape((1,)).astype(tokenized_text.dtype)
    emb_t = embedded_text.transpose(1, 0, 2)
    ph3 = placeholder_embedding[None]
    out_t = pl.pallas_call(
        _select_body,
        grid_spec=pltpu.PrefetchScalarGridSpec(
            num_scalar_prefetch=1,
            grid=(N,),
            in_specs=[
                pl.BlockSpec((B, N), lambda j, pt: (0, 0)),
                pl.BlockSpec((1, B, D), lambda j, pt: (j, 0, 0)),
                pl.BlockSpec((1, 1, D), lambda j, pt: (0, 0, 0)),
            ],
            out_specs=pl.BlockSpec((1, B, D), lambda j, pt: (j, 0, 0)),
        ),
        out_shape=jax.ShapeDtypeStruct((N, B, D), jnp.float32),
        compiler_params=pltpu.CompilerParams(
            dimension_semantics=("arbitrary",),
        ),
    )(pt, tokenized_text, emb_t, ph3)
    return out_t.transpose(1, 0, 2)
